# Initial kernel scaffold; baseline (speedup 1.0000x reference)
import jax
import jax.numpy as jnp
from jax.experimental import pallas as pl


def kernel(x, edge_index, batch, params):
    raise NotImplementedError("write your pallas kernel here")


# speedup vs baseline: 10.5106x; 10.5106x over previous
import functools

import jax
import jax.numpy as jnp
from jax import lax
from jax.experimental import pallas as pl
from jax.experimental.pallas import tpu as pltpu
from jax.experimental.pallas import tpu_sc as plsc

NG = 100
F = 128
N = NG * F
E = 409600
HID = 64
NGP = 104
TRI = F * (F - 1) // 2

NC = 2
NS = 16
NW = NC * NS
EPT = E // NW
CH = 128
NCH = EPT // CH
RPT = N // NS
FPT = NG * TRI // NW
FPTP = 25600
FCH = FPTP // CH

_HI = lax.Precision.HIGHEST


def _dot(a, b):
  return lax.dot_general(a, b, (((1,), (0,)), ((), ())), precision=_HI)


def _deg_body(d_hbm, zeros_hbm, ones_hbm, out_hbm, zbuf, ones_v, didx, hist):
  c = lax.axis_index("c")
  t = lax.axis_index("s")

  pltpu.sync_copy(zeros_hbm, zbuf)
  pltpu.sync_copy(ones_hbm, ones_v)

  row0 = t * RPT
  pltpu.sync_copy(zbuf, hist.at[pl.ds(row0, RPT)])
  plsc.subcore_barrier()

  base = (c * NS + t) * EPT

  def body(k, carry):
    pltpu.sync_copy(d_hbm.at[pl.ds(base + k * CH, CH)], didx)
    pltpu.sync_copy(ones_v, hist.at[didx], add=True)
    return carry
  lax.fori_loop(0, NCH, body, 0)

  plsc.subcore_barrier()
  pltpu.sync_copy(hist.at[pl.ds(row0, RPT)], zbuf)
  pltpu.sync_copy(zbuf, out_hbm.at[c, pl.ds(row0, RPT)])


def _edge_body(hwp_hbm, s_hbm, d_hbm, zeros_hbm, out_hbm, zbuf, rows, sidx,
               didx, agg, sem):
  c = lax.axis_index("c")
  t = lax.axis_index("s")

  pltpu.sync_copy(zeros_hbm, zbuf)

  row0 = t * RPT
  for p in range(RPT // 200):
    pltpu.sync_copy(zbuf, agg.at[pl.ds(row0 + p * 200, 200)])
  plsc.subcore_barrier()

  base = (c * NS + t) * EPT

  def body(k, carry):
    off = base + k * CH
    pltpu.sync_copy(s_hbm.at[pl.ds(off, CH)], sidx)
    pltpu.async_copy(hwp_hbm.at[sidx], rows, sem).wait()
    pltpu.sync_copy(d_hbm.at[pl.ds(off, CH)], didx)
    pltpu.sync_copy(rows, agg.at[didx], add=True)
    return carry
  lax.fori_loop(0, NCH, body, 0)

  plsc.subcore_barrier()
  for p in range(RPT // 200):
    pltpu.sync_copy(agg.at[pl.ds(row0 + p * 200, 200)], zbuf)
    pltpu.sync_copy(zbuf, out_hbm.at[c, pl.ds(row0 + p * 200, 200)])


def _feat_body(xflat_hbm, fidx_hbm, out_hbm, fi, row, sem):
  w = lax.axis_index("c") * NS + lax.axis_index("s")

  def body(k, carry):
    pltpu.sync_copy(fidx_hbm.at[w, pl.ds(k * CH, CH)], fi)
    pltpu.async_copy(xflat_hbm.at[fi], row, sem).wait()
    pltpu.sync_copy(row, out_hbm.at[w, pl.ds(k * CH, CH)])
    return carry
  lax.fori_loop(0, FCH, body, 0)


@functools.cache
def _sc_kernels():
  mesh = plsc.VectorSubcoreMesh(
      core_axis_name="c", subcore_axis_name="s",
      num_cores=NC, num_subcores=NS)
  params = pltpu.CompilerParams(use_tc_tiling_on_sc=False)
  deg = pl.kernel(
      _deg_body,
      compiler_params=params,
      out_type=jax.ShapeDtypeStruct((NC, N, 8), jnp.float32),
      mesh=mesh,
      scratch_types=[
          pltpu.VMEM((RPT, 8), jnp.float32),
          pltpu.VMEM((CH, 8), jnp.float32),
          pltpu.VMEM((CH,), jnp.int32),
          pltpu.VMEM_SHARED((N, 8), jnp.float32),
      ])
  edge = pl.kernel(
      _edge_body,
      compiler_params=params,
      out_type=jax.ShapeDtypeStruct((NC, N, HID), jnp.float32),
      mesh=mesh,
      scratch_types=[
          pltpu.VMEM((200, HID), jnp.float32),
          pltpu.VMEM((CH, HID), jnp.float32),
          pltpu.VMEM((CH,), jnp.int32),
          pltpu.VMEM((CH,), jnp.int32),
          pltpu.VMEM_SHARED((N, HID), jnp.float32),
          pltpu.SemaphoreType.DMA,
      ])
  feat = pl.kernel(
      _feat_body,
      compiler_params=params,
      out_type=jax.ShapeDtypeStruct((NW, FPTP), jnp.float32),
      mesh=mesh,
      scratch_types=[
          pltpu.VMEM((CH,), jnp.int32),
          pltpu.VMEM((CH,), jnp.float32),
          pltpu.SemaphoreType.DMA,
      ])
  return deg, edge, feat


def _prep_body(x_ref, w0_ref, p_ref, hwp_ref, dinv_ref):
  deg = p_ref[0, :, 0:1] + p_ref[1, :, 0:1] + 1.0
  dinv = lax.rsqrt(deg)
  hw = _dot(x_ref[...], w0_ref[...])
  hwp_ref[...] = hw * dinv
  dinv_ref[...] = jnp.broadcast_to(dinv, (F, 8))


def _mid_body(p_ref, hwp_ref, dinv_ref, b_ref, w_ref, hwp1_ref, m_ref):
  dinv = dinv_ref[:, 0:1]
  x1 = jnp.tanh(dinv * (p_ref[0] + p_ref[1] + hwp_ref[...]) + b_ref[...])
  m_ref[...] = jnp.sum(x1, axis=0, keepdims=True)[None] * (1.0 / F)
  hwp1_ref[...] = _dot(x1, w_ref[...]) * dinv


def _last_body(p_ref, hwp_ref, dinv_ref, b_ref, m_ref):
  dinv = dinv_ref[:, 0:1]
  x2 = jnp.tanh(dinv * (p_ref[0] + p_ref[1] + hwp_ref[...]) + b_ref[...])
  m_ref[...] = jnp.sum(x2, axis=0, keepdims=True)[None] * (1.0 / F)


def _head_body(feat, m1, m2, bng, bnb, bnhg, bnhb, w0a, w0b, b0, g0, be0,
               w1, b1, g1, be1, w2, b2, g2, be2, w3, b3, out):
  rows = lax.broadcasted_iota(jnp.int32, (NGP, 1), 0)
  mask = rows < NG
  inv = 1.0 / NG

  def stats(vm):
    m = jnp.sum(vm, axis=0, keepdims=True) * inv
    var = jnp.sum(vm * vm, axis=0, keepdims=True) * inv - m * m
    return m, lax.rsqrt(var + 1e-5)

  f = feat[...]
  m, r = stats(f)
  fn = (f - m) * r * bng[...] + bnb[...]
  h = jnp.concatenate([m1[...], m2[...]], axis=1)
  mh, rh = stats(h)
  hn = (h - mh) * rh * bnhg[...] + bnhb[...]

  def bstage(z, g, b):
    zm = jnp.where(mask, z, 0.0)
    mz, rz = stats(zm)
    return jax.nn.relu((z - mz) * rz * g[...] + b[...])

  z = bstage(_dot(fn, w0a[...]) + _dot(hn, w0b[...]) + b0[...], g0, be0)
  z = bstage(_dot(z, w1[...]) + b1[...], g1, be1)
  z = bstage(_dot(z, w2[...]) + b2[...], g2, be2)
  out[...] = _dot(z, w3[...]) + b3[...]


def _full(shape):
  return pl.BlockSpec(shape, lambda i: tuple(0 for _ in shape))


_prep = pl.pallas_call(
    _prep_body,
    grid=(NG,),
    in_specs=[
        pl.BlockSpec((F, F), lambda i: (i, 0)),
        pl.BlockSpec((F, HID), lambda i: (0, 0)),
        pl.BlockSpec((NC, F, 8), lambda i: (0, i, 0)),
    ],
    out_specs=[
        pl.BlockSpec((F, HID), lambda i: (i, 0)),
        pl.BlockSpec((F, 8), lambda i: (i, 0)),
    ],
    out_shape=[
        jax.ShapeDtypeStruct((N, HID), jnp.float32),
        jax.ShapeDtypeStruct((N, 8), jnp.float32),
    ],
)

_mid = pl.pallas_call(
    _mid_body,
    grid=(NG,),
    in_specs=[
        pl.BlockSpec((NC, F, HID), lambda i: (0, i, 0)),
        pl.BlockSpec((F, HID), lambda i: (i, 0)),
        pl.BlockSpec((F, 8), lambda i: (i, 0)),
        pl.BlockSpec((1, HID), lambda i: (0, 0)),
        pl.BlockSpec((HID, HID), lambda i: (0, 0)),
    ],
    out_specs=[
        pl.BlockSpec((F, HID), lambda i: (i, 0)),
        pl.BlockSpec((1, 1, HID), lambda i: (i, 0, 0)),
    ],
    out_shape=[
        jax.ShapeDtypeStruct((N, HID), jnp.float32),
        jax.ShapeDtypeStruct((NG, 1, HID), jnp.float32),
    ],
)

_last = pl.pallas_call(
    _last_body,
    grid=(NG,),
    in_specs=[
        pl.BlockSpec((NC, F, HID), lambda i: (0, i, 0)),
        pl.BlockSpec((F, HID), lambda i: (i, 0)),
        pl.BlockSpec((F, 8), lambda i: (i, 0)),
        pl.BlockSpec((1, HID), lambda i: (0, 0)),
    ],
    out_specs=pl.BlockSpec((1, 1, HID), lambda i: (i, 0, 0)),
    out_shape=jax.ShapeDtypeStruct((NG, 1, HID), jnp.float32),
)

_head = pl.pallas_call(
    _head_body,
    out_shape=jax.ShapeDtypeStruct((NGP, 2), jnp.float32),
)


def _feat_indices():
  iu, ju = jnp.triu_indices(F, k=1)
  off = (iu * F + ju).astype(jnp.int32)
  gidx = (jnp.arange(NG, dtype=jnp.int32)[:, None] * (F * F)
          + off[None, :]).reshape(NW, FPT)
  return jnp.pad(gidx, ((0, 0), (0, FPTP - FPT)))


def kernel(x, edge_index, batch, params):
  del batch
  src = edge_index[0]
  dst = edge_index[1]

  deg_k, edge_k, feat_k = _sc_kernels()
  zeros8 = jnp.zeros((RPT, 8), jnp.float32)
  ones8 = jnp.ones((CH, 8), jnp.float32)
  zeros64 = jnp.zeros((200, HID), jnp.float32)
  degp = deg_k(dst, zeros8, ones8)
  featw = feat_k(x.reshape(-1), _feat_indices())
  feat = featw[:, :FPT].reshape(NG, TRI)

  hwp0, dinv8 = _prep(x, params["conv0_W"], degp)
  agg0 = edge_k(hwp0, src, dst, zeros64)
  hwp1, m1 = _mid(agg0, hwp0, dinv8, params["conv0_b"].reshape(1, HID),
                  params["conv1_W"])
  agg1 = edge_k(hwp1, src, dst, zeros64)
  m2 = _last(agg1, hwp1, dinv8, params["conv1_b"].reshape(1, HID))

  pad = ((0, NGP - NG), (0, 0))
  r = lambda v: v.reshape(1, -1)
  out = _head(
      jnp.pad(feat, pad),
      jnp.pad(m1.reshape(NG, HID), pad),
      jnp.pad(m2.reshape(NG, HID), pad),
      r(params["bn_g"]), r(params["bn_b"]),
      r(params["bnh_g"]), r(params["bnh_b"]),
      params["mlp0_W"][:TRI], params["mlp0_W"][TRI:], r(params["mlp0_b"]),
      r(params["mbn0_g"]), r(params["mbn0_b"]),
      params["mlp1_W"], r(params["mlp1_b"]),
      r(params["mbn1_g"]), r(params["mbn1_b"]),
      params["mlp2_W"], r(params["mlp2_b"]),
      r(params["mbn2_g"]), r(params["mbn2_b"]),
      params["mlp3_W"], r(params["mlp3_b"]))
  return out[:NG]


# speedup vs baseline: 17.4379x; 1.6591x over previous
import functools

import jax
import jax.numpy as jnp
from jax import lax
from jax.experimental import pallas as pl
from jax.experimental.pallas import tpu as pltpu
from jax.experimental.pallas import tpu_sc as plsc

NG = 100
F = 128
N = NG * F
E = 409600
HID = 64
NGP = 104
TRI = F * (F - 1) // 2

NC = 2
NS = 16
NW = NC * NS
EPT = E // NW
CH = 128
NCH = EPT // CH
RPT = N // NS
NBUF = 4

_HI = lax.Precision.HIGHEST


def _dot(a, b):
  return lax.dot_general(a, b, (((1,), (0,)), ((), ())), precision=_HI)


def _deg_body(d_hbm, zeros_hbm, ones_hbm, out_hbm, zbuf, ones_v, didx, hist,
              *sems):
  c = lax.axis_index("c")
  t = lax.axis_index("s")

  pltpu.sync_copy(zeros_hbm, zbuf)
  pltpu.sync_copy(ones_hbm, ones_v)

  row0 = t * RPT
  pltpu.sync_copy(zbuf, hist.at[pl.ds(row0, RPT)])
  plsc.subcore_barrier()

  base = (c * NS + t) * EPT

  def body(m, carry):
    off = base + m * (NBUF * CH)
    descs = []
    for b in range(NBUF):
      pltpu.sync_copy(d_hbm.at[pl.ds(off + b * CH, CH)], didx.at[b])
      descs.append(
          pltpu.async_copy(ones_v, hist.at[didx.at[b]], sems[b], add=True))
    for d in descs:
      d.wait()
    return carry
  lax.fori_loop(0, NCH // NBUF, body, 0)

  plsc.subcore_barrier()
  pltpu.sync_copy(hist.at[pl.ds(row0, RPT)], zbuf)
  pltpu.sync_copy(zbuf, out_hbm.at[c, pl.ds(row0, RPT)])


def _edge_body(hwp_hbm, s_hbm, d_hbm, zeros_hbm, out_hbm, zbuf, rows, sidx,
               didx, agg, *sems):
  c = lax.axis_index("c")
  t = lax.axis_index("s")

  pltpu.sync_copy(zeros_hbm, zbuf)

  row0 = t * RPT
  for p in range(RPT // 200):
    pltpu.sync_copy(zbuf, agg.at[pl.ds(row0 + p * 200, 200)])
  plsc.subcore_barrier()

  base = (c * NS + t) * EPT

  def body(m, carry):
    off = base + m * (NBUF * CH)
    descs = []
    for b in range(NBUF):
      pltpu.sync_copy(s_hbm.at[pl.ds(off + b * CH, CH)], sidx.at[b])
      descs.append(
          pltpu.async_copy(hwp_hbm.at[sidx.at[b]], rows.at[b], sems[b]))
    for b in range(NBUF):
      pltpu.sync_copy(d_hbm.at[pl.ds(off + b * CH, CH)], didx.at[b])
      descs[b].wait()
      pltpu.sync_copy(rows.at[b], agg.at[didx.at[b]], add=True)
    return carry
  lax.fori_loop(0, NCH // NBUF, body, 0)

  plsc.subcore_barrier()
  for p in range(RPT // 200):
    pltpu.sync_copy(agg.at[pl.ds(row0 + p * 200, 200)], zbuf)
    pltpu.sync_copy(zbuf, out_hbm.at[c, pl.ds(row0 + p * 200, 200)])


@functools.cache
def _sc_kernels():
  mesh = plsc.VectorSubcoreMesh(
      core_axis_name="c", subcore_axis_name="s",
      num_cores=NC, num_subcores=NS)
  params = pltpu.CompilerParams(use_tc_tiling_on_sc=False)
  deg = pl.kernel(
      _deg_body,
      compiler_params=params,
      out_type=jax.ShapeDtypeStruct((NC, N, 8), jnp.float32),
      mesh=mesh,
      scratch_types=[
          pltpu.VMEM((RPT, 8), jnp.float32),
          pltpu.VMEM((CH, 8), jnp.float32),
          pltpu.VMEM((NBUF, CH), jnp.int32),
          pltpu.VMEM_SHARED((N, 8), jnp.float32),
      ] + [pltpu.SemaphoreType.DMA] * NBUF)
  edge = pl.kernel(
      _edge_body,
      compiler_params=params,
      out_type=jax.ShapeDtypeStruct((NC, N, HID), jnp.float32),
      mesh=mesh,
      scratch_types=[
          pltpu.VMEM((200, HID), jnp.float32),
          pltpu.VMEM((NBUF, CH, HID), jnp.float32),
          pltpu.VMEM((NBUF, CH), jnp.int32),
          pltpu.VMEM((NBUF, CH), jnp.int32),
          pltpu.VMEM_SHARED((N, HID), jnp.float32),
      ] + [pltpu.SemaphoreType.DMA] * NBUF)
  return deg, edge


def _featc_body(x_ref, f_ref):
  off = 0
  for i in range(F - 1):
    seg = F - 1 - i
    f_ref[0, 0, pl.ds(off, seg)] = x_ref[i, pl.ds(i + 1, seg)]
    off += seg


def _prep_body(x_ref, w0_ref, p_ref, hwp_ref, dinv_ref):
  deg = p_ref[0, :, 0:1] + p_ref[1, :, 0:1] + 1.0
  dinv = lax.rsqrt(deg)
  hw = _dot(x_ref[...], w0_ref[...])
  hwp_ref[...] = hw * dinv
  dinv_ref[...] = jnp.broadcast_to(dinv, (F, 8))


def _mid_body(p_ref, hwp_ref, dinv_ref, b_ref, w_ref, hwp1_ref, m_ref):
  dinv = dinv_ref[:, 0:1]
  x1 = jnp.tanh(dinv * (p_ref[0] + p_ref[1] + hwp_ref[...]) + b_ref[...])
  m_ref[...] = jnp.sum(x1, axis=0, keepdims=True)[None] * (1.0 / F)
  hwp1_ref[...] = _dot(x1, w_ref[...]) * dinv


def _last_body(p_ref, hwp_ref, dinv_ref, b_ref, m_ref):
  dinv = dinv_ref[:, 0:1]
  x2 = jnp.tanh(dinv * (p_ref[0] + p_ref[1] + hwp_ref[...]) + b_ref[...])
  m_ref[...] = jnp.sum(x2, axis=0, keepdims=True)[None] * (1.0 / F)


def _head_body(feat, m1, m2, bng, bnb, bnhg, bnhb, w0a, w0b, b0, g0, be0,
               w1, b1, g1, be1, w2, b2, g2, be2, w3, b3, out):
  rows = lax.broadcasted_iota(jnp.int32, (NGP, 1), 0)
  mask = rows < NG
  inv = 1.0 / NG

  def stats(vm):
    m = jnp.sum(vm, axis=0, keepdims=True) * inv
    var = jnp.sum(vm * vm, axis=0, keepdims=True) * inv - m * m
    return m, lax.rsqrt(var + 1e-5)

  f = feat[...]
  m, r = stats(f)
  fn = (f - m) * r * bng[...] + bnb[...]
  h = jnp.concatenate([m1[...], m2[...]], axis=1)
  mh, rh = stats(h)
  hn = (h - mh) * rh * bnhg[...] + bnhb[...]

  def bstage(z, g, b):
    zm = jnp.where(mask, z, 0.0)
    mz, rz = stats(zm)
    return jax.nn.relu((z - mz) * rz * g[...] + b[...])

  z = bstage(_dot(fn, w0a[...]) + _dot(hn, w0b[...]) + b0[...], g0, be0)
  z = bstage(_dot(z, w1[...]) + b1[...], g1, be1)
  z = bstage(_dot(z, w2[...]) + b2[...], g2, be2)
  out[...] = _dot(z, w3[...]) + b3[...]


_featc = pl.pallas_call(
    _featc_body,
    grid=(NG,),
    in_specs=[pl.BlockSpec((F, F), lambda i: (i, 0))],
    out_specs=pl.BlockSpec((1, 1, TRI), lambda i: (i, 0, 0)),
    out_shape=jax.ShapeDtypeStruct((NG, 1, TRI), jnp.float32),
)

_prep = pl.pallas_call(
    _prep_body,
    grid=(NG,),
    in_specs=[
        pl.BlockSpec((F, F), lambda i: (i, 0)),
        pl.BlockSpec((F, HID), lambda i: (0, 0)),
        pl.BlockSpec((NC, F, 8), lambda i: (0, i, 0)),
    ],
    out_specs=[
        pl.BlockSpec((F, HID), lambda i: (i, 0)),
        pl.BlockSpec((F, 8), lambda i: (i, 0)),
    ],
    out_shape=[
        jax.ShapeDtypeStruct((N, HID), jnp.float32),
        jax.ShapeDtypeStruct((N, 8), jnp.float32),
    ],
)

_mid = pl.pallas_call(
    _mid_body,
    grid=(NG,),
    in_specs=[
        pl.BlockSpec((NC, F, HID), lambda i: (0, i, 0)),
        pl.BlockSpec((F, HID), lambda i: (i, 0)),
        pl.BlockSpec((F, 8), lambda i: (i, 0)),
        pl.BlockSpec((1, HID), lambda i: (0, 0)),
        pl.BlockSpec((HID, HID), lambda i: (0, 0)),
    ],
    out_specs=[
        pl.BlockSpec((F, HID), lambda i: (i, 0)),
        pl.BlockSpec((1, 1, HID), lambda i: (i, 0, 0)),
    ],
    out_shape=[
        jax.ShapeDtypeStruct((N, HID), jnp.float32),
        jax.ShapeDtypeStruct((NG, 1, HID), jnp.float32),
    ],
)

_last = pl.pallas_call(
    _last_body,
    grid=(NG,),
    in_specs=[
        pl.BlockSpec((NC, F, HID), lambda i: (0, i, 0)),
        pl.BlockSpec((F, HID), lambda i: (i, 0)),
        pl.BlockSpec((F, 8), lambda i: (i, 0)),
        pl.BlockSpec((1, HID), lambda i: (0, 0)),
    ],
    out_specs=pl.BlockSpec((1, 1, HID), lambda i: (i, 0, 0)),
    out_shape=jax.ShapeDtypeStruct((NG, 1, HID), jnp.float32),
)

_head = pl.pallas_call(
    _head_body,
    out_shape=jax.ShapeDtypeStruct((NGP, 2), jnp.float32),
)


def kernel(x, edge_index, batch, params):
  del batch
  src = edge_index[0]
  dst = edge_index[1]

  deg_k, edge_k = _sc_kernels()
  zeros8 = jnp.zeros((RPT, 8), jnp.float32)
  ones8 = jnp.ones((CH, 8), jnp.float32)
  zeros64 = jnp.zeros((200, HID), jnp.float32)
  degp = deg_k(dst, zeros8, ones8)
  feat = _featc(x).reshape(NG, TRI)

  hwp0, dinv8 = _prep(x, params["conv0_W"], degp)
  agg0 = edge_k(hwp0, src, dst, zeros64)
  hwp1, m1 = _mid(agg0, hwp0, dinv8, params["conv0_b"].reshape(1, HID),
                  params["conv1_W"])
  agg1 = edge_k(hwp1, src, dst, zeros64)
  m2 = _last(agg1, hwp1, dinv8, params["conv1_b"].reshape(1, HID))

  pad = ((0, NGP - NG), (0, 0))
  r = lambda v: v.reshape(1, -1)
  out = _head(
      jnp.pad(feat, pad),
      jnp.pad(m1.reshape(NG, HID), pad),
      jnp.pad(m2.reshape(NG, HID), pad),
      r(params["bn_g"]), r(params["bn_b"]),
      r(params["bnh_g"]), r(params["bnh_b"]),
      params["mlp0_W"][:TRI], params["mlp0_W"][TRI:], r(params["mlp0_b"]),
      r(params["mbn0_g"]), r(params["mbn0_b"]),
      params["mlp1_W"], r(params["mlp1_b"]),
      r(params["mbn1_g"]), r(params["mbn1_b"]),
      params["mlp2_W"], r(params["mlp2_b"]),
      r(params["mbn2_g"]), r(params["mbn2_b"]),
      params["mlp3_W"], r(params["mlp3_b"]))
  return out[:NG]


# speedup vs baseline: 20.8077x; 1.1932x over previous
import functools

import jax
import jax.numpy as jnp
from jax import lax
from jax.experimental import pallas as pl
from jax.experimental.pallas import tpu as pltpu
from jax.experimental.pallas import tpu_sc as plsc

NG = 100
F = 128
N = NG * F
E = 409600
HID = 64
NGP = 104
TRI = F * (F - 1) // 2

NC = 2
NS = 16
NW = NC * NS
EPT = E // NW
CH = 128
NCH = EPT // CH
RPT = N // NS
NBUF = 5

_HI = lax.Precision.HIGHEST


def _dot(a, b):
  return lax.dot_general(a, b, (((1,), (0,)), ((), ())), precision=_HI)


def _deg_body(d_hbm, zeros_hbm, ones_hbm, out_hbm, zbuf, ones_v, didx, hist,
              *sems):
  c = lax.axis_index("c")
  t = lax.axis_index("s")

  pltpu.sync_copy(zeros_hbm, zbuf)
  pltpu.sync_copy(ones_hbm, ones_v)

  row0 = t * RPT
  pltpu.sync_copy(zbuf, hist.at[pl.ds(row0, RPT)])
  plsc.subcore_barrier()

  base = (c * NS + t) * EPT

  def body(m, carry):
    off = base + m * (NBUF * CH)
    descs = []
    for b in range(NBUF):
      pltpu.sync_copy(d_hbm.at[pl.ds(off + b * CH, CH)], didx.at[b])
      descs.append(
          pltpu.async_copy(ones_v, hist.at[didx.at[b]], sems[b], add=True))
    for d in descs:
      d.wait()
    return carry
  lax.fori_loop(0, NCH // NBUF, body, 0)

  plsc.subcore_barrier()
  pltpu.sync_copy(hist.at[pl.ds(row0, RPT)], zbuf)
  pltpu.sync_copy(zbuf, out_hbm.at[c, pl.ds(row0, RPT)])


def _edge_body(hwp_hbm, s_hbm, d_hbm, zeros_hbm, out_hbm, zbuf, rows, sidx,
               didx, agg, *sems):
  c = lax.axis_index("c")
  t = lax.axis_index("s")

  pltpu.sync_copy(zeros_hbm, zbuf)

  row0 = t * RPT
  for p in range(RPT // 200):
    pltpu.sync_copy(zbuf, agg.at[pl.ds(row0 + p * 200, 200)])
  plsc.subcore_barrier()

  base = (c * NS + t) * EPT

  gsems = sems[:NBUF]
  ssems = sems[NBUF:]

  def body(m, carry):
    off = base + m * (NBUF * CH)
    gd = []
    for b in range(NBUF):
      pltpu.sync_copy(s_hbm.at[pl.ds(off + b * CH, CH)], sidx.at[b])
      gd.append(
          pltpu.async_copy(hwp_hbm.at[sidx.at[b]], rows.at[b], gsems[b]))
    sd = []
    for b in range(NBUF):
      pltpu.sync_copy(d_hbm.at[pl.ds(off + b * CH, CH)], didx.at[b])
      gd[b].wait()
      sd.append(
          pltpu.async_copy(rows.at[b], agg.at[didx.at[b]], ssems[b],
                           add=True))
    for d in sd:
      d.wait()
    return carry
  lax.fori_loop(0, NCH // NBUF, body, 0)

  plsc.subcore_barrier()
  for p in range(RPT // 200):
    pltpu.sync_copy(agg.at[pl.ds(row0 + p * 200, 200)], zbuf)
    pltpu.sync_copy(zbuf, out_hbm.at[c, pl.ds(row0 + p * 200, 200)])


@functools.cache
def _sc_kernels():
  mesh = plsc.VectorSubcoreMesh(
      core_axis_name="c", subcore_axis_name="s",
      num_cores=NC, num_subcores=NS)
  params = pltpu.CompilerParams(use_tc_tiling_on_sc=False)
  deg = pl.kernel(
      _deg_body,
      compiler_params=params,
      out_type=jax.ShapeDtypeStruct((NC, N, 8), jnp.float32),
      mesh=mesh,
      scratch_types=[
          pltpu.VMEM((RPT, 8), jnp.float32),
          pltpu.VMEM((CH, 8), jnp.float32),
          pltpu.VMEM((NBUF, CH), jnp.int32),
          pltpu.VMEM_SHARED((N, 8), jnp.float32),
      ] + [pltpu.SemaphoreType.DMA] * NBUF)
  edge = pl.kernel(
      _edge_body,
      compiler_params=params,
      out_type=jax.ShapeDtypeStruct((NC, N, HID), jnp.float32),
      mesh=mesh,
      scratch_types=[
          pltpu.VMEM((200, HID), jnp.float32),
          pltpu.VMEM((NBUF, CH, HID), jnp.float32),
          pltpu.VMEM((NBUF, CH), jnp.int32),
          pltpu.VMEM((NBUF, CH), jnp.int32),
          pltpu.VMEM_SHARED((N, HID), jnp.float32),
      ] + [pltpu.SemaphoreType.DMA] * (2 * NBUF))
  return deg, edge


def _prepa_body(x_ref, w0_ref, hw_ref, f_ref):
  hw_ref[...] = _dot(x_ref[...], w0_ref[...])
  off = 0
  for i in range(F - 1):
    seg = F - 1 - i
    f_ref[0, 0, pl.ds(off, seg)] = x_ref[i, pl.ds(i + 1, seg)]
    off += seg


def _prepb_body(hw_ref, p_ref, hwp_ref, dinv_ref):
  deg = p_ref[0, :, 0:1] + p_ref[1, :, 0:1] + 1.0
  dinv = lax.rsqrt(deg)
  hwp_ref[...] = hw_ref[...] * dinv
  dinv_ref[...] = jnp.broadcast_to(dinv, (F, 8))


def _mid_body(p_ref, hwp_ref, dinv_ref, b_ref, w_ref, hwp1_ref, m_ref):
  dinv = dinv_ref[:, 0:1]
  x1 = jnp.tanh(dinv * (p_ref[0] + p_ref[1] + hwp_ref[...]) + b_ref[...])
  m_ref[...] = jnp.sum(x1, axis=0, keepdims=True)[None] * (1.0 / F)
  hwp1_ref[...] = _dot(x1, w_ref[...]) * dinv


def _tail_body(q, hwp1, dinv8, b1c, feat, m1, bng, bnb, bnhg, bnhb,
               w0a, w0b, b0, g0, be0, w1, b1m, g1, be1, w2, b2m, g2, be2,
               w3, b3m, out):
  dinv = dinv8[:, 0:1]
  x2 = jnp.tanh(dinv * (q[0] + q[1] + hwp1[...]) + b1c[...])
  m2 = jnp.sum(x2.reshape(NG, F, HID), axis=1) * (1.0 / F)

  rows = lax.broadcasted_iota(jnp.int32, (NGP, 1), 0)
  mask = rows < NG
  inv = 1.0 / NG

  def stats(vm):
    m = jnp.sum(vm, axis=0, keepdims=True) * inv
    var = jnp.sum(vm * vm, axis=0, keepdims=True) * inv - m * m
    return m, lax.rsqrt(var + 1e-5)

  zp = jnp.zeros((NGP - NG, HID), jnp.float32)
  f = jnp.concatenate(
      [feat[...], jnp.zeros((NGP - NG, TRI), jnp.float32)], axis=0)
  m, r = stats(f)
  fn = (f - m) * r * bng[...] + bnb[...]
  h = jnp.concatenate([
      jnp.concatenate([m1[...], zp], axis=0),
      jnp.concatenate([m2, zp], axis=0)], axis=1)
  mh, rh = stats(h)
  hn = (h - mh) * rh * bnhg[...] + bnhb[...]

  def bstage(z, g, b):
    zm = jnp.where(mask, z, 0.0)
    mz, rz = stats(zm)
    return jax.nn.relu((z - mz) * rz * g[...] + b[...])

  z = bstage(_dot(fn, w0a[...]) + _dot(hn, w0b[...]) + b0[...], g0, be0)
  z = bstage(_dot(z, w1[...]) + b1m[...], g1, be1)
  z = bstage(_dot(z, w2[...]) + b2m[...], g2, be2)
  out[...] = _dot(z, w3[...]) + b3m[...]


_prepa = pl.pallas_call(
    _prepa_body,
    grid=(NG,),
    in_specs=[
        pl.BlockSpec((F, F), lambda i: (i, 0)),
        pl.BlockSpec((F, HID), lambda i: (0, 0)),
    ],
    out_specs=[
        pl.BlockSpec((F, HID), lambda i: (i, 0)),
        pl.BlockSpec((1, 1, TRI), lambda i: (i, 0, 0)),
    ],
    out_shape=[
        jax.ShapeDtypeStruct((N, HID), jnp.float32),
        jax.ShapeDtypeStruct((NG, 1, TRI), jnp.float32),
    ],
)

_prepb = pl.pallas_call(
    _prepb_body,
    grid=(NG,),
    in_specs=[
        pl.BlockSpec((F, HID), lambda i: (i, 0)),
        pl.BlockSpec((NC, F, 8), lambda i: (0, i, 0)),
    ],
    out_specs=[
        pl.BlockSpec((F, HID), lambda i: (i, 0)),
        pl.BlockSpec((F, 8), lambda i: (i, 0)),
    ],
    out_shape=[
        jax.ShapeDtypeStruct((N, HID), jnp.float32),
        jax.ShapeDtypeStruct((N, 8), jnp.float32),
    ],
)

_mid = pl.pallas_call(
    _mid_body,
    grid=(NG,),
    in_specs=[
        pl.BlockSpec((NC, F, HID), lambda i: (0, i, 0)),
        pl.BlockSpec((F, HID), lambda i: (i, 0)),
        pl.BlockSpec((F, 8), lambda i: (i, 0)),
        pl.BlockSpec((1, HID), lambda i: (0, 0)),
        pl.BlockSpec((HID, HID), lambda i: (0, 0)),
    ],
    out_specs=[
        pl.BlockSpec((F, HID), lambda i: (i, 0)),
        pl.BlockSpec((1, 1, HID), lambda i: (i, 0, 0)),
    ],
    out_shape=[
        jax.ShapeDtypeStruct((N, HID), jnp.float32),
        jax.ShapeDtypeStruct((NG, 1, HID), jnp.float32),
    ],
)

_tail = pl.pallas_call(
    _tail_body,
    out_shape=jax.ShapeDtypeStruct((NGP, 2), jnp.float32),
)


def kernel(x, edge_index, batch, params):
  del batch
  src = edge_index[0]
  dst = edge_index[1]

  deg_k, edge_k = _sc_kernels()
  zeros8 = jnp.zeros((RPT, 8), jnp.float32)
  ones8 = jnp.ones((CH, 8), jnp.float32)
  zeros64 = jnp.zeros((200, HID), jnp.float32)
  degp = deg_k(dst, zeros8, ones8)
  hw0, feat3 = _prepa(x, params["conv0_W"])
  feat = feat3.reshape(NG, TRI)

  hwp0, dinv8 = _prepb(hw0, degp)
  agg0 = edge_k(hwp0, src, dst, zeros64)
  hwp1, m1 = _mid(agg0, hwp0, dinv8, params["conv0_b"].reshape(1, HID),
                  params["conv1_W"])
  agg1 = edge_k(hwp1, src, dst, zeros64)

  r = lambda v: v.reshape(1, -1)
  out = _tail(
      agg1, hwp1, dinv8, params["conv1_b"].reshape(1, HID),
      feat, m1.reshape(NG, HID),
      r(params["bn_g"]), r(params["bn_b"]),
      r(params["bnh_g"]), r(params["bnh_b"]),
      params["mlp0_W"][:TRI], params["mlp0_W"][TRI:], r(params["mlp0_b"]),
      r(params["mbn0_g"]), r(params["mbn0_b"]),
      params["mlp1_W"], r(params["mlp1_b"]),
      r(params["mbn1_g"]), r(params["mbn1_b"]),
      params["mlp2_W"], r(params["mlp2_b"]),
      r(params["mbn2_g"]), r(params["mbn2_b"]),
      params["mlp3_W"], r(params["mlp3_b"]))
  return out[:NG]


# speedup vs baseline: 23.1498x; 1.1126x over previous
import functools

import jax
import jax.numpy as jnp
from jax import lax
from jax.experimental import pallas as pl
from jax.experimental.pallas import tpu as pltpu
from jax.experimental.pallas import tpu_sc as plsc

NG = 100
F = 128
N = NG * F
E = 409600
HID = 64
NGP = 104
TRI = F * (F - 1) // 2

NC = 2
NS = 16
NW = NC * NS
EPT = E // NW
CH = 128
NCH = EPT // CH
RPT = N // NS
NBUF = 5

_HI = lax.Precision.HIGHEST


def _dot(a, b):
  return lax.dot_general(a, b, (((1,), (0,)), ((), ())), precision=_HI)


def _deg_body(d_hbm, zeros_hbm, ones_hbm, out_hbm, zbuf, ones_v, didx, hist,
              *sems):
  c = lax.axis_index("c")
  t = lax.axis_index("s")

  pltpu.sync_copy(zeros_hbm, zbuf)
  pltpu.sync_copy(ones_hbm, ones_v)
  crow = (c * NS + t) * NCH
  pltpu.sync_copy(d_hbm.at[pl.ds(crow, NCH)], didx)

  row0 = t * RPT
  pltpu.sync_copy(zbuf, hist.at[pl.ds(row0, RPT)])
  plsc.subcore_barrier()

  def body(m, carry):
    descs = []
    for b in range(NBUF):
      descs.append(
          pltpu.async_copy(ones_v, hist.at[didx.at[m * NBUF + b]], sems[b],
                           add=True))
    for d in descs:
      d.wait()
    return carry
  lax.fori_loop(0, NCH // NBUF, body, 0)

  plsc.subcore_barrier()
  pltpu.sync_copy(hist.at[pl.ds(row0, RPT)], zbuf)
  pltpu.sync_copy(zbuf, out_hbm.at[c, pl.ds(row0, RPT)])


def _edge_body(hwp_hbm, s_hbm, d_hbm, zeros_hbm, out_hbm, zbuf, rows, sidx,
               didx, agg, *sems):
  c = lax.axis_index("c")
  t = lax.axis_index("s")

  pltpu.sync_copy(zeros_hbm, zbuf)
  crow = (c * NS + t) * NCH
  pltpu.sync_copy(s_hbm.at[pl.ds(crow, NCH)], sidx)
  pltpu.sync_copy(d_hbm.at[pl.ds(crow, NCH)], didx)

  row0 = t * RPT
  for p in range(RPT // 200):
    pltpu.sync_copy(zbuf, agg.at[pl.ds(row0 + p * 200, 200)])
  plsc.subcore_barrier()

  gsems = sems[:NBUF]
  ssems = sems[NBUF:]

  def body(m, carry):
    gd = []
    for b in range(NBUF):
      gd.append(
          pltpu.async_copy(hwp_hbm.at[sidx.at[m * NBUF + b]], rows.at[b],
                           gsems[b]))
    sd = []
    for b in range(NBUF):
      gd[b].wait()
      sd.append(
          pltpu.async_copy(rows.at[b], agg.at[didx.at[m * NBUF + b]],
                           ssems[b], add=True))
    for d in sd:
      d.wait()
    return carry
  lax.fori_loop(0, NCH // NBUF, body, 0)

  plsc.subcore_barrier()
  for p in range(RPT // 200):
    pltpu.sync_copy(agg.at[pl.ds(row0 + p * 200, 200)], zbuf)
    pltpu.sync_copy(zbuf, out_hbm.at[c, pl.ds(row0 + p * 200, 200)])


@functools.cache
def _sc_kernels():
  mesh = plsc.VectorSubcoreMesh(
      core_axis_name="c", subcore_axis_name="s",
      num_cores=NC, num_subcores=NS)
  params = pltpu.CompilerParams(use_tc_tiling_on_sc=False)
  deg = pl.kernel(
      _deg_body,
      compiler_params=params,
      out_type=jax.ShapeDtypeStruct((NC, N, 8), jnp.float32),
      mesh=mesh,
      scratch_types=[
          pltpu.VMEM((RPT, 8), jnp.float32),
          pltpu.VMEM((CH, 8), jnp.float32),
          pltpu.VMEM((NCH, CH), jnp.int32),
          pltpu.VMEM_SHARED((N, 8), jnp.float32),
      ] + [pltpu.SemaphoreType.DMA] * NBUF)
  edge = pl.kernel(
      _edge_body,
      compiler_params=params,
      out_type=jax.ShapeDtypeStruct((NC, N, HID), jnp.float32),
      mesh=mesh,
      scratch_types=[
          pltpu.VMEM((200, HID), jnp.float32),
          pltpu.VMEM((NBUF, CH, HID), jnp.float32),
          pltpu.VMEM((NCH, CH), jnp.int32),
          pltpu.VMEM((NCH, CH), jnp.int32),
          pltpu.VMEM_SHARED((N, HID), jnp.float32),
      ] + [pltpu.SemaphoreType.DMA] * (2 * NBUF))
  return deg, edge


def _prepa_body(x_ref, w0_ref, hw_ref, f_ref):
  hw_ref[...] = _dot(x_ref[...], w0_ref[...])
  off = 0
  for i in range(F - 1):
    seg = F - 1 - i
    f_ref[0, 0, pl.ds(off, seg)] = x_ref[i, pl.ds(i + 1, seg)]
    off += seg


def _prepb_body(hw_ref, p_ref, hwp_ref, dinv_ref):
  deg = p_ref[0, :, 0:1] + p_ref[1, :, 0:1] + 1.0
  dinv = lax.rsqrt(deg)
  hwp_ref[...] = hw_ref[...] * dinv
  dinv_ref[...] = jnp.broadcast_to(dinv, (F, 8))


def _mid_body(p_ref, hwp_ref, dinv_ref, b_ref, w_ref, hwp1_ref, m_ref):
  dinv = dinv_ref[:, 0:1]
  x1 = jnp.tanh(dinv * (p_ref[0] + p_ref[1] + hwp_ref[...]) + b_ref[...])
  m_ref[...] = jnp.sum(x1, axis=0, keepdims=True)[None] * (1.0 / F)
  hwp1_ref[...] = _dot(x1, w_ref[...]) * dinv


def _tail_body(q, hwp1, dinv8, b1c, feat, m1, bng, bnb, bnhg, bnhb,
               w0a, w0b, b0, g0, be0, w1, b1m, g1, be1, w2, b2m, g2, be2,
               w3, b3m, out):
  dinv = dinv8[:, 0:1]
  x2 = jnp.tanh(dinv * (q[0] + q[1] + hwp1[...]) + b1c[...])
  m2 = jnp.sum(x2.reshape(NG, F, HID), axis=1) * (1.0 / F)

  rows = lax.broadcasted_iota(jnp.int32, (NGP, 1), 0)
  mask = rows < NG
  inv = 1.0 / NG

  def stats(vm):
    m = jnp.sum(vm, axis=0, keepdims=True) * inv
    var = jnp.sum(vm * vm, axis=0, keepdims=True) * inv - m * m
    return m, lax.rsqrt(var + 1e-5)

  zp = jnp.zeros((NGP - NG, HID), jnp.float32)
  f = jnp.concatenate(
      [feat[...], jnp.zeros((NGP - NG, TRI), jnp.float32)], axis=0)
  m, r = stats(f)
  fn = (f - m) * r * bng[...] + bnb[...]
  h = jnp.concatenate([
      jnp.concatenate([m1[...], zp], axis=0),
      jnp.concatenate([m2, zp], axis=0)], axis=1)
  mh, rh = stats(h)
  hn = (h - mh) * rh * bnhg[...] + bnhb[...]

  def bstage(z, g, b):
    zm = jnp.where(mask, z, 0.0)
    mz, rz = stats(zm)
    return jax.nn.relu((z - mz) * rz * g[...] + b[...])

  z = bstage(_dot(fn, w0a[...]) + _dot(hn, w0b[...]) + b0[...], g0, be0)
  z = bstage(_dot(z, w1[...]) + b1m[...], g1, be1)
  z = bstage(_dot(z, w2[...]) + b2m[...], g2, be2)
  out[...] = _dot(z, w3[...]) + b3m[...]


_prepa = pl.pallas_call(
    _prepa_body,
    grid=(NG,),
    in_specs=[
        pl.BlockSpec((F, F), lambda i: (i, 0)),
        pl.BlockSpec((F, HID), lambda i: (0, 0)),
    ],
    out_specs=[
        pl.BlockSpec((F, HID), lambda i: (i, 0)),
        pl.BlockSpec((1, 1, TRI), lambda i: (i, 0, 0)),
    ],
    out_shape=[
        jax.ShapeDtypeStruct((N, HID), jnp.float32),
        jax.ShapeDtypeStruct((NG, 1, TRI), jnp.float32),
    ],
)

_prepb = pl.pallas_call(
    _prepb_body,
    grid=(NG,),
    in_specs=[
        pl.BlockSpec((F, HID), lambda i: (i, 0)),
        pl.BlockSpec((NC, F, 8), lambda i: (0, i, 0)),
    ],
    out_specs=[
        pl.BlockSpec((F, HID), lambda i: (i, 0)),
        pl.BlockSpec((F, 8), lambda i: (i, 0)),
    ],
    out_shape=[
        jax.ShapeDtypeStruct((N, HID), jnp.float32),
        jax.ShapeDtypeStruct((N, 8), jnp.float32),
    ],
)

_mid = pl.pallas_call(
    _mid_body,
    grid=(NG,),
    in_specs=[
        pl.BlockSpec((NC, F, HID), lambda i: (0, i, 0)),
        pl.BlockSpec((F, HID), lambda i: (i, 0)),
        pl.BlockSpec((F, 8), lambda i: (i, 0)),
        pl.BlockSpec((1, HID), lambda i: (0, 0)),
        pl.BlockSpec((HID, HID), lambda i: (0, 0)),
    ],
    out_specs=[
        pl.BlockSpec((F, HID), lambda i: (i, 0)),
        pl.BlockSpec((1, 1, HID), lambda i: (i, 0, 0)),
    ],
    out_shape=[
        jax.ShapeDtypeStruct((N, HID), jnp.float32),
        jax.ShapeDtypeStruct((NG, 1, HID), jnp.float32),
    ],
)

_tail = pl.pallas_call(
    _tail_body,
    out_shape=jax.ShapeDtypeStruct((NGP, 2), jnp.float32),
)


def kernel(x, edge_index, batch, params):
  del batch
  src = edge_index[0]
  dst = edge_index[1]

  deg_k, edge_k = _sc_kernels()
  src = src.reshape(E // CH, CH)
  dst = dst.reshape(E // CH, CH)
  zeros8 = jnp.zeros((RPT, 8), jnp.float32)
  ones8 = jnp.ones((CH, 8), jnp.float32)
  zeros64 = jnp.zeros((200, HID), jnp.float32)
  degp = deg_k(dst, zeros8, ones8)
  hw0, feat3 = _prepa(x, params["conv0_W"])
  feat = feat3.reshape(NG, TRI)

  hwp0, dinv8 = _prepb(hw0, degp)
  agg0 = edge_k(hwp0, src, dst, zeros64)
  hwp1, m1 = _mid(agg0, hwp0, dinv8, params["conv0_b"].reshape(1, HID),
                  params["conv1_W"])
  agg1 = edge_k(hwp1, src, dst, zeros64)

  r = lambda v: v.reshape(1, -1)
  out = _tail(
      agg1, hwp1, dinv8, params["conv1_b"].reshape(1, HID),
      feat, m1.reshape(NG, HID),
      r(params["bn_g"]), r(params["bn_b"]),
      r(params["bnh_g"]), r(params["bnh_b"]),
      params["mlp0_W"][:TRI], params["mlp0_W"][TRI:], r(params["mlp0_b"]),
      r(params["mbn0_g"]), r(params["mbn0_b"]),
      params["mlp1_W"], r(params["mlp1_b"]),
      r(params["mbn1_g"]), r(params["mbn1_b"]),
      params["mlp2_W"], r(params["mlp2_b"]),
      r(params["mbn2_g"]), r(params["mbn2_b"]),
      params["mlp3_W"], r(params["mlp3_b"]))
  return out[:NG]


# speedup vs baseline: 23.5834x; 1.0187x over previous
import functools

import jax
import jax.numpy as jnp
from jax import lax
from jax.experimental import pallas as pl
from jax.experimental.pallas import tpu as pltpu
from jax.experimental.pallas import tpu_sc as plsc

NG = 100
F = 128
N = NG * F
E = 409600
HID = 64
NGP = 104
TRI = F * (F - 1) // 2

NC = 2
NS = 16
NW = NC * NS
EPT = E // NW
CH = 128
NCH = EPT // CH
RPT = N // NS
NBUF = 5

_HI = lax.Precision.HIGHEST


def _dot(a, b):
  return lax.dot_general(a, b, (((1,), (0,)), ((), ())))


def _deg_body(d_hbm, zeros_hbm, ones_hbm, out_hbm, zbuf, ones_v, didx, hist,
              *sems):
  c = lax.axis_index("c")
  t = lax.axis_index("s")

  pltpu.sync_copy(zeros_hbm, zbuf)
  pltpu.sync_copy(ones_hbm, ones_v)
  crow = (c * NS + t) * NCH
  pltpu.sync_copy(d_hbm.at[pl.ds(crow, NCH)], didx)

  row0 = t * RPT
  pltpu.sync_copy(zbuf, hist.at[pl.ds(row0, RPT)])
  plsc.subcore_barrier()

  def body(m, carry):
    descs = []
    for b in range(NBUF):
      descs.append(
          pltpu.async_copy(ones_v, hist.at[didx.at[m * NBUF + b]], sems[b],
                           add=True))
    for d in descs:
      d.wait()
    return carry
  lax.fori_loop(0, NCH // NBUF, body, 0)

  plsc.subcore_barrier()
  pltpu.sync_copy(hist.at[pl.ds(row0, RPT)], zbuf)
  pltpu.sync_copy(zbuf, out_hbm.at[c, pl.ds(row0, RPT)])


def _edge_body(hwp_hbm, s_hbm, d_hbm, zeros_hbm, out_hbm, zbuf, rows, sidx,
               didx, agg, *sems):
  c = lax.axis_index("c")
  t = lax.axis_index("s")

  pltpu.sync_copy(zeros_hbm, zbuf)
  crow = (c * NS + t) * NCH
  pltpu.sync_copy(s_hbm.at[pl.ds(crow, NCH)], sidx)
  pltpu.sync_copy(d_hbm.at[pl.ds(crow, NCH)], didx)

  row0 = t * RPT
  for p in range(RPT // 200):
    pltpu.sync_copy(zbuf, agg.at[pl.ds(row0 + p * 200, 200)])
  plsc.subcore_barrier()

  gsems = sems[:NBUF]
  ssems = sems[NBUF:]

  def body(m, carry):
    gd = []
    for b in range(NBUF):
      gd.append(
          pltpu.async_copy(hwp_hbm.at[sidx.at[m * NBUF + b]], rows.at[b],
                           gsems[b]))
    sd = []
    for b in range(NBUF):
      gd[b].wait()
      sd.append(
          pltpu.async_copy(rows.at[b], agg.at[didx.at[m * NBUF + b]],
                           ssems[b], add=True))
    for d in sd:
      d.wait()
    return carry
  lax.fori_loop(0, NCH // NBUF, body, 0)

  plsc.subcore_barrier()
  for p in range(RPT // 200):
    pltpu.sync_copy(agg.at[pl.ds(row0 + p * 200, 200)], zbuf)
    pltpu.sync_copy(zbuf, out_hbm.at[c, pl.ds(row0 + p * 200, 200)])


@functools.cache
def _sc_kernels():
  mesh = plsc.VectorSubcoreMesh(
      core_axis_name="c", subcore_axis_name="s",
      num_cores=NC, num_subcores=NS)
  params = pltpu.CompilerParams(use_tc_tiling_on_sc=False)
  deg = pl.kernel(
      _deg_body,
      compiler_params=params,
      out_type=jax.ShapeDtypeStruct((NC, N, 8), jnp.float32),
      mesh=mesh,
      scratch_types=[
          pltpu.VMEM((RPT, 8), jnp.float32),
          pltpu.VMEM((CH, 8), jnp.float32),
          pltpu.VMEM((NCH, CH), jnp.int32),
          pltpu.VMEM_SHARED((N, 8), jnp.float32),
      ] + [pltpu.SemaphoreType.DMA] * NBUF)
  edge = pl.kernel(
      _edge_body,
      compiler_params=params,
      out_type=jax.ShapeDtypeStruct((NC, N, HID), jnp.float32),
      mesh=mesh,
      scratch_types=[
          pltpu.VMEM((200, HID), jnp.float32),
          pltpu.VMEM((NBUF, CH, HID), jnp.float32),
          pltpu.VMEM((NCH, CH), jnp.int32),
          pltpu.VMEM((NCH, CH), jnp.int32),
          pltpu.VMEM_SHARED((N, HID), jnp.float32),
      ] + [pltpu.SemaphoreType.DMA] * (2 * NBUF))
  return deg, edge


def _prepa_body(x_ref, w0_ref, hw_ref, f_ref):
  hw_ref[...] = _dot(x_ref[...], w0_ref[...])
  off = 0
  for i in range(F - 1):
    seg = F - 1 - i
    f_ref[0, 0, pl.ds(off, seg)] = x_ref[i, pl.ds(i + 1, seg)]
    off += seg


def _prepb_body(hw_ref, p_ref, hwp_ref, dinv_ref):
  deg = p_ref[0, :, 0:1] + p_ref[1, :, 0:1] + 1.0
  dinv = lax.rsqrt(deg)
  hwp_ref[...] = hw_ref[...] * dinv
  dinv_ref[...] = jnp.broadcast_to(dinv, (F, 8))


def _mid_body(p_ref, hwp_ref, dinv_ref, b_ref, w_ref, hwp1_ref, m_ref):
  dinv = dinv_ref[:, 0:1]
  x1 = jnp.tanh(dinv * (p_ref[0] + p_ref[1] + hwp_ref[...]) + b_ref[...])
  m_ref[...] = jnp.sum(x1, axis=0, keepdims=True)[None] * (1.0 / F)
  hwp1_ref[...] = _dot(x1, w_ref[...]) * dinv


def _tail_body(q, hwp1, dinv8, b1c, feat, m1, bng, bnb, bnhg, bnhb,
               w0a, w0b, b0, g0, be0, w1, b1m, g1, be1, w2, b2m, g2, be2,
               w3, b3m, out):
  dinv = dinv8[:, 0:1]
  x2 = jnp.tanh(dinv * (q[0] + q[1] + hwp1[...]) + b1c[...])
  m2 = jnp.sum(x2.reshape(NG, F, HID), axis=1) * (1.0 / F)

  rows = lax.broadcasted_iota(jnp.int32, (NGP, 1), 0)
  mask = rows < NG
  inv = 1.0 / NG

  def stats(vm):
    m = jnp.sum(vm, axis=0, keepdims=True) * inv
    var = jnp.sum(vm * vm, axis=0, keepdims=True) * inv - m * m
    return m, lax.rsqrt(var + 1e-5)

  zp = jnp.zeros((NGP - NG, HID), jnp.float32)
  f = jnp.concatenate(
      [feat[...], jnp.zeros((NGP - NG, TRI), jnp.float32)], axis=0)
  m, r = stats(f)
  fn = (f - m) * r * bng[...] + bnb[...]
  h = jnp.concatenate([
      jnp.concatenate([m1[...], zp], axis=0),
      jnp.concatenate([m2, zp], axis=0)], axis=1)
  mh, rh = stats(h)
  hn = (h - mh) * rh * bnhg[...] + bnhb[...]

  def bstage(z, g, b):
    zm = jnp.where(mask, z, 0.0)
    mz, rz = stats(zm)
    return jax.nn.relu((z - mz) * rz * g[...] + b[...])

  z = bstage(_dot(fn, w0a[...]) + _dot(hn, w0b[...]) + b0[...], g0, be0)
  z = bstage(_dot(z, w1[...]) + b1m[...], g1, be1)
  z = bstage(_dot(z, w2[...]) + b2m[...], g2, be2)
  out[...] = _dot(z, w3[...]) + b3m[...]


_prepa = pl.pallas_call(
    _prepa_body,
    grid=(NG,),
    in_specs=[
        pl.BlockSpec((F, F), lambda i: (i, 0)),
        pl.BlockSpec((F, HID), lambda i: (0, 0)),
    ],
    out_specs=[
        pl.BlockSpec((F, HID), lambda i: (i, 0)),
        pl.BlockSpec((1, 1, TRI), lambda i: (i, 0, 0)),
    ],
    out_shape=[
        jax.ShapeDtypeStruct((N, HID), jnp.float32),
        jax.ShapeDtypeStruct((NG, 1, TRI), jnp.float32),
    ],
)

_prepb = pl.pallas_call(
    _prepb_body,
    grid=(NG,),
    in_specs=[
        pl.BlockSpec((F, HID), lambda i: (i, 0)),
        pl.BlockSpec((NC, F, 8), lambda i: (0, i, 0)),
    ],
    out_specs=[
        pl.BlockSpec((F, HID), lambda i: (i, 0)),
        pl.BlockSpec((F, 8), lambda i: (i, 0)),
    ],
    out_shape=[
        jax.ShapeDtypeStruct((N, HID), jnp.float32),
        jax.ShapeDtypeStruct((N, 8), jnp.float32),
    ],
)

_mid = pl.pallas_call(
    _mid_body,
    grid=(NG,),
    in_specs=[
        pl.BlockSpec((NC, F, HID), lambda i: (0, i, 0)),
        pl.BlockSpec((F, HID), lambda i: (i, 0)),
        pl.BlockSpec((F, 8), lambda i: (i, 0)),
        pl.BlockSpec((1, HID), lambda i: (0, 0)),
        pl.BlockSpec((HID, HID), lambda i: (0, 0)),
    ],
    out_specs=[
        pl.BlockSpec((F, HID), lambda i: (i, 0)),
        pl.BlockSpec((1, 1, HID), lambda i: (i, 0, 0)),
    ],
    out_shape=[
        jax.ShapeDtypeStruct((N, HID), jnp.float32),
        jax.ShapeDtypeStruct((NG, 1, HID), jnp.float32),
    ],
)

_tail = pl.pallas_call(
    _tail_body,
    out_shape=jax.ShapeDtypeStruct((NGP, 2), jnp.float32),
)


def kernel(x, edge_index, batch, params):
  del batch
  src = edge_index[0]
  dst = edge_index[1]

  deg_k, edge_k = _sc_kernels()
  src = src.reshape(E // CH, CH)
  dst = dst.reshape(E // CH, CH)
  zeros8 = jnp.zeros((RPT, 8), jnp.float32)
  ones8 = jnp.ones((CH, 8), jnp.float32)
  zeros64 = jnp.zeros((200, HID), jnp.float32)
  degp = deg_k(dst, zeros8, ones8)
  hw0, feat3 = _prepa(x, params["conv0_W"])
  feat = feat3.reshape(NG, TRI)

  hwp0, dinv8 = _prepb(hw0, degp)
  agg0 = edge_k(hwp0, src, dst, zeros64)
  hwp1, m1 = _mid(agg0, hwp0, dinv8, params["conv0_b"].reshape(1, HID),
                  params["conv1_W"])
  agg1 = edge_k(hwp1, src, dst, zeros64)

  r = lambda v: v.reshape(1, -1)
  out = _tail(
      agg1, hwp1, dinv8, params["conv1_b"].reshape(1, HID),
      feat, m1.reshape(NG, HID),
      r(params["bn_g"]), r(params["bn_b"]),
      r(params["bnh_g"]), r(params["bnh_b"]),
      params["mlp0_W"][:TRI], params["mlp0_W"][TRI:], r(params["mlp0_b"]),
      r(params["mbn0_g"]), r(params["mbn0_b"]),
      params["mlp1_W"], r(params["mlp1_b"]),
      r(params["mbn1_g"]), r(params["mbn1_b"]),
      params["mlp2_W"], r(params["mlp2_b"]),
      r(params["mbn2_g"]), r(params["mbn2_b"]),
      params["mlp3_W"], r(params["mlp3_b"]))
  return out[:NG]


# speedup vs baseline: 33.6167x; 1.4254x over previous
import functools

import jax
import jax.numpy as jnp
from jax import lax
from jax.experimental import pallas as pl
from jax.experimental.pallas import tpu as pltpu
from jax.experimental.pallas import tpu_sc as plsc

NG = 100
F = 128
N = NG * F
E = 409600
HID = 64
NGP = 104
TRI = F * (F - 1) // 2
HIDDEN = 128

NC = 2
NS = 16
NW = NC * NS
EPT = E // NW
CH = 128
NCH = EPT // CH
RPT = N // NS
NBUF = 5

_HI = lax.Precision.HIGHEST


def _dot(a, b):
  return lax.dot_general(a, b, (((1,), (0,)), ((), ())))


def _deg_body(d_hbm, zeros_hbm, ones_hbm, out_hbm, zbuf, ones_v, didx, hist,
              *sems):
  c = lax.axis_index("c")
  t = lax.axis_index("s")

  pltpu.sync_copy(zeros_hbm, zbuf)
  pltpu.sync_copy(ones_hbm, ones_v)
  crow = (c * NS + t) * NCH
  pltpu.sync_copy(d_hbm.at[pl.ds(crow, NCH)], didx)

  row0 = t * RPT
  pltpu.sync_copy(zbuf, hist.at[pl.ds(row0, RPT)])
  plsc.subcore_barrier()

  def body(m, carry):
    descs = []
    for b in range(NBUF):
      descs.append(
          pltpu.async_copy(ones_v, hist.at[didx.at[m * NBUF + b]], sems[b],
                           add=True))
    for d in descs:
      d.wait()
    return carry
  lax.fori_loop(0, NCH // NBUF, body, 0)

  plsc.subcore_barrier()
  pltpu.sync_copy(hist.at[pl.ds(row0, RPT)], zbuf)
  pltpu.sync_copy(zbuf, out_hbm.at[c, pl.ds(row0, RPT)])


def _edge_body(hwp_hbm, s_hbm, d_hbm, zeros_hbm, out_hbm, zbuf, rows, sidx,
               didx, agg, *sems):
  c = lax.axis_index("c")
  t = lax.axis_index("s")

  pltpu.sync_copy(zeros_hbm, zbuf)
  crow = (c * NS + t) * NCH
  pltpu.sync_copy(s_hbm.at[pl.ds(crow, NCH)], sidx)
  pltpu.sync_copy(d_hbm.at[pl.ds(crow, NCH)], didx)

  row0 = t * RPT
  for p in range(RPT // 200):
    pltpu.sync_copy(zbuf, agg.at[pl.ds(row0 + p * 200, 200)])
  plsc.subcore_barrier()

  gsems = sems[:NBUF]
  ssems = sems[NBUF:]

  def body(m, carry):
    gd = []
    for b in range(NBUF):
      gd.append(
          pltpu.async_copy(hwp_hbm.at[sidx.at[m * NBUF + b]], rows.at[b],
                           gsems[b]))
    sd = []
    for b in range(NBUF):
      gd[b].wait()
      sd.append(
          pltpu.async_copy(rows.at[b], agg.at[didx.at[m * NBUF + b]],
                           ssems[b], add=True))
    for d in sd:
      d.wait()
    return carry
  lax.fori_loop(0, NCH // NBUF, body, 0)

  plsc.subcore_barrier()
  for p in range(RPT // 200):
    pltpu.sync_copy(agg.at[pl.ds(row0 + p * 200, 200)], zbuf)
    pltpu.sync_copy(zbuf, out_hbm.at[c, pl.ds(row0 + p * 200, 200)])


@functools.cache
def _sc_kernels():
  mesh = plsc.VectorSubcoreMesh(
      core_axis_name="c", subcore_axis_name="s",
      num_cores=NC, num_subcores=NS)
  params = pltpu.CompilerParams(use_tc_tiling_on_sc=False)
  deg = pl.kernel(
      _deg_body,
      compiler_params=params,
      out_type=jax.ShapeDtypeStruct((NC, N, 8), jnp.float32),
      mesh=mesh,
      scratch_types=[
          pltpu.VMEM((RPT, 8), jnp.float32),
          pltpu.VMEM((CH, 8), jnp.float32),
          pltpu.VMEM((NCH, CH), jnp.int32),
          pltpu.VMEM_SHARED((N, 8), jnp.float32),
      ] + [pltpu.SemaphoreType.DMA] * NBUF)
  edge = pl.kernel(
      _edge_body,
      compiler_params=params,
      out_type=jax.ShapeDtypeStruct((NC, N, HID), jnp.float32),
      mesh=mesh,
      scratch_types=[
          pltpu.VMEM((200, HID), jnp.float32),
          pltpu.VMEM((NBUF, CH, HID), jnp.float32),
          pltpu.VMEM((NCH, CH), jnp.int32),
          pltpu.VMEM((NCH, CH), jnp.int32),
          pltpu.VMEM_SHARED((N, HID), jnp.float32),
      ] + [pltpu.SemaphoreType.DMA] * (2 * NBUF))
  return deg, edge


GB = 10
RB = GB * F


def _prepa_body(x_ref, w0_ref, hw_ref):
  hw_ref[...] = _dot(x_ref[...], w0_ref[...])


def _prepb_body(hw_ref, p_ref, hwp_ref, dinv_ref):
  deg = p_ref[0, :, 0:1] + p_ref[1, :, 0:1] + 1.0
  dinv = lax.rsqrt(deg)
  hwp_ref[...] = hw_ref[...] * dinv
  dinv_ref[...] = jnp.broadcast_to(dinv, (RB, 8))


def _mid_body(p_ref, hwp_ref, dinv_ref, b_ref, w_ref, hwp1_ref, m_ref):
  dinv = dinv_ref[:, 0:1]
  x1 = jnp.tanh(dinv * (p_ref[0] + p_ref[1] + hwp_ref[...]) + b_ref[...])
  m_ref[...] = jnp.sum(x1.reshape(GB, F, HID), axis=1)[None] * (1.0 / F)
  hwp1_ref[...] = _dot(x1, w_ref[...]) * dinv


def _expand_body(x_ref, w8_ref, g_ref, b_ref, wp_ref, k_ref, ge_ref, be_ref):
  x3 = x_ref[...].reshape(NG, F, F)
  inv = 1.0 / NG
  m = jnp.sum(x3, axis=0) * inv
  var = jnp.sum(x3 * x3, axis=0) * inv - m * m
  r = lax.rsqrt(var + 1e-5)

  ge_ref[...] = jnp.zeros((F, F), jnp.float32)
  be_ref[...] = jnp.zeros((F, F), jnp.float32)
  off = 0
  for i in range(F - 1):
    seg = F - 1 - i
    ge_ref[i, pl.ds(i + 1, seg)] = g_ref[0, pl.ds(off, seg)]
    be_ref[i, pl.ds(i + 1, seg)] = b_ref[0, pl.ds(off, seg)]
    off += seg
  ge = ge_ref[...]
  be = be_ref[...]
  S = r * ge
  T = be - m * S
  St = S.T

  wp_ref[...] = jnp.zeros((F * F, HIDDEN), jnp.float32)
  k = jnp.zeros((1, HIDDEN), jnp.float32)
  off = 0
  for i in range(F - 1):
    seg = F - 1 - i
    wp_ref[pl.ds(i * F + i + 1, seg), :] = (
        w8_ref[pl.ds(off, seg), :] * St[i + 1:, i:i + 1])
    k = k + _dot(T[i:i + 1, i + 1:], w8_ref[pl.ds(off, seg), :])
    off += seg
  k_ref[...] = k


def _tail_body(q, hwp1, dinv8, b1c, xflat, wp, kc, m1, bnhg, bnhb,
               w0b, b0, g0, be0, w1, b1m, g1, be1, w2, b2m, g2, be2,
               w3, b3m, out):
  dinv = dinv8[:, 0:1]
  x2 = jnp.tanh(dinv * (q[0] + q[1] + hwp1[...]) + b1c[...])
  m2 = jnp.sum(x2.reshape(NG, F, HID), axis=1) * (1.0 / F)

  rows = lax.broadcasted_iota(jnp.int32, (NGP, 1), 0)
  mask = rows < NG
  inv = 1.0 / NG

  def stats(vm):
    m = jnp.sum(vm, axis=0, keepdims=True) * inv
    var = jnp.sum(vm * vm, axis=0, keepdims=True) * inv - m * m
    return m, lax.rsqrt(var + 1e-5)

  zp = jnp.zeros((NGP - NG, HID), jnp.float32)
  h = jnp.concatenate([
      jnp.concatenate([m1[...], zp], axis=0),
      jnp.concatenate([m2, zp], axis=0)], axis=1)
  mh, rh = stats(h)
  hn = (h - mh) * rh * bnhg[...] + bnhb[...]

  def bstage(z, g, b):
    zm = jnp.where(mask, z, 0.0)
    mz, rz = stats(zm)
    return jax.nn.relu((z - mz) * rz * g[...] + b[...])

  z = bstage(
      _dot(xflat[...], wp[...]) + kc[...] + _dot(hn, w0b[...]) + b0[...],
      g0, be0)
  z = bstage(_dot(z, w1[...]) + b1m[...], g1, be1)
  z = bstage(_dot(z, w2[...]) + b2m[...], g2, be2)
  out[...] = _dot(z, w3[...]) + b3m[...]


_prepa = pl.pallas_call(
    _prepa_body,
    grid=(N // RB,),
    in_specs=[
        pl.BlockSpec((RB, F), lambda i: (i, 0)),
        pl.BlockSpec((F, HID), lambda i: (0, 0)),
    ],
    out_specs=pl.BlockSpec((RB, HID), lambda i: (i, 0)),
    out_shape=jax.ShapeDtypeStruct((N, HID), jnp.float32),
)

_prepb = pl.pallas_call(
    _prepb_body,
    grid=(N // RB,),
    in_specs=[
        pl.BlockSpec((RB, HID), lambda i: (i, 0)),
        pl.BlockSpec((NC, RB, 8), lambda i: (0, i, 0)),
    ],
    out_specs=[
        pl.BlockSpec((RB, HID), lambda i: (i, 0)),
        pl.BlockSpec((RB, 8), lambda i: (i, 0)),
    ],
    out_shape=[
        jax.ShapeDtypeStruct((N, HID), jnp.float32),
        jax.ShapeDtypeStruct((N, 8), jnp.float32),
    ],
)

_mid = pl.pallas_call(
    _mid_body,
    grid=(N // RB,),
    in_specs=[
        pl.BlockSpec((NC, RB, HID), lambda i: (0, i, 0)),
        pl.BlockSpec((RB, HID), lambda i: (i, 0)),
        pl.BlockSpec((RB, 8), lambda i: (i, 0)),
        pl.BlockSpec((1, HID), lambda i: (0, 0)),
        pl.BlockSpec((HID, HID), lambda i: (0, 0)),
    ],
    out_specs=[
        pl.BlockSpec((RB, HID), lambda i: (i, 0)),
        pl.BlockSpec((1, GB, HID), lambda i: (i, 0, 0)),
    ],
    out_shape=[
        jax.ShapeDtypeStruct((N, HID), jnp.float32),
        jax.ShapeDtypeStruct((NG // GB, GB, HID), jnp.float32),
    ],
)

_expand = pl.pallas_call(
    _expand_body,
    out_shape=[
        jax.ShapeDtypeStruct((F * F, HIDDEN), jnp.float32),
        jax.ShapeDtypeStruct((1, HIDDEN), jnp.float32),
    ],
    scratch_shapes=[
        pltpu.VMEM((F, F), jnp.float32),
        pltpu.VMEM((F, F), jnp.float32),
    ],
)

_tail = pl.pallas_call(
    _tail_body,
    out_shape=jax.ShapeDtypeStruct((NGP, 2), jnp.float32),
)


def kernel(x, edge_index, batch, params):
  del batch
  src = edge_index[0]
  dst = edge_index[1]

  deg_k, edge_k = _sc_kernels()
  src = src.reshape(E // CH, CH)
  dst = dst.reshape(E // CH, CH)
  zeros8 = jnp.zeros((RPT, 8), jnp.float32)
  ones8 = jnp.ones((CH, 8), jnp.float32)
  zeros64 = jnp.zeros((200, HID), jnp.float32)
  r = lambda v: v.reshape(1, -1)
  degp = deg_k(dst, zeros8, ones8)
  hw0 = _prepa(x, params["conv0_W"])
  wp, kc = _expand(x, params["mlp0_W"][:TRI], r(params["bn_g"]),
                   r(params["bn_b"]))
  xflat = jnp.pad(x.reshape(NG, F * F), ((0, NGP - NG), (0, 0)))

  hwp0, dinv8 = _prepb(hw0, degp)
  agg0 = edge_k(hwp0, src, dst, zeros64)
  hwp1, m1 = _mid(agg0, hwp0, dinv8, params["conv0_b"].reshape(1, HID),
                  params["conv1_W"])
  agg1 = edge_k(hwp1, src, dst, zeros64)

  out = _tail(
      agg1, hwp1, dinv8, params["conv1_b"].reshape(1, HID),
      xflat, wp, kc, m1.reshape(NG, HID),
      r(params["bnh_g"]), r(params["bnh_b"]),
      params["mlp0_W"][TRI:], r(params["mlp0_b"]),
      r(params["mbn0_g"]), r(params["mbn0_b"]),
      params["mlp1_W"], r(params["mlp1_b"]),
      r(params["mbn1_g"]), r(params["mbn1_b"]),
      params["mlp2_W"], r(params["mlp2_b"]),
      r(params["mbn2_g"]), r(params["mbn2_b"]),
      params["mlp3_W"], r(params["mlp3_b"]))
  return out[:NG]
